# Initial kernel scaffold; baseline (speedup 1.0000x reference)
import jax
import jax.numpy as jnp
from jax.experimental import pallas as pl


def kernel(index, length, data):
    raise NotImplementedError("write your pallas kernel here")


# speedup vs baseline: 6.8385x; 6.8385x over previous
import functools

import jax
import jax.numpy as jnp
from jax import lax
from jax.experimental import pallas as pl
from jax.experimental.pallas import tpu as pltpu
from jax.experimental.pallas import tpu_sc as plsc

_WINDOW = 200
_NUM_CORES = 2
_NUM_SUBCORES = 16
_NW = _NUM_CORES * _NUM_SUBCORES
_LANES = 16


@functools.partial(jax.jit, static_argnums=(2, 3, 4))
def _sc_window_gather(table_flat, start, batch, channels, b_per_w):
    ext_words = table_flat.shape[0]
    out_words = _WINDOW * channels
    mesh = plsc.VectorSubcoreMesh(
        core_axis_name="c",
        subcore_axis_name="s",
        num_cores=_NUM_CORES,
        num_subcores=_NUM_SUBCORES,
    )

    @functools.partial(
        pl.kernel,
        mesh=mesh,
        out_type=jax.ShapeDtypeStruct((batch, out_words), jnp.float32),
        scratch_types=[
            pltpu.VMEM((b_per_w,), jnp.int32),
            pltpu.VMEM((ext_words,), jnp.float32),
            pltpu.SemaphoreType.DMA,
            pltpu.SemaphoreType.DMA,
        ],
        compiler_params=pltpu.CompilerParams(use_tc_tiling_on_sc=False),
    )
    def k(table_hbm, start_hbm, out_hbm, idx_v, table_v, sem_idx, sem_out):
        wid = lax.axis_index("s") * _NUM_CORES + lax.axis_index("c")
        base = wid * b_per_w
        idx_cp = pltpu.make_async_copy(
            start_hbm.at[pl.ds(base, b_per_w)], idx_v, sem_idx
        )
        tbl_cp = pltpu.make_async_copy(table_hbm, table_v, sem_out)
        idx_cp.start()
        tbl_cp.start()
        idx_cp.wait()
        tbl_cp.wait()

        def fire(g, carry):
            vec = idx_v[pl.ds(g * _LANES, _LANES)] * channels
            for lane in range(_LANES):
                pltpu.make_async_copy(
                    table_v.at[pl.ds(pl.multiple_of(vec[lane], 8), out_words)],
                    out_hbm.at[base + g * _LANES + lane],
                    sem_out,
                ).start()
            return carry

        lax.fori_loop(0, b_per_w // _LANES, fire, 0)

        def drain(b, carry):
            pltpu.make_async_copy(
                table_v.at[pl.ds(0, out_words)], out_hbm.at[base + b], sem_out
            ).wait()
            return carry

        lax.fori_loop(0, b_per_w, drain, 0)

    return k(table_flat, start)


def kernel(index, length, data):
    cycle_len, channels = data.shape
    batch = index.shape[0]
    start = jnp.asarray(
        (index.astype(jnp.int32) + (length - _WINDOW)) % cycle_len, jnp.int32
    )
    table_flat = jnp.concatenate([data, data[:_WINDOW]], axis=0).reshape(-1)
    out = _sc_window_gather(table_flat, start, batch, channels, batch // _NW)
    return out.reshape(batch, _WINDOW, channels)
